# baseline (speedup 1.0000x reference)
import functools

import jax
import jax.numpy as jnp
from jax import lax
from jax.experimental import pallas as pl
from jax.experimental.pallas import tpu as pltpu
from jax.experimental.pallas import tpu_sc as plsc

N = 10000
E = 320000
D = 128
G = 64

CH = 128
ROWS = E // CH
RPT = 160
RLAST = ROWS - 15 * RPT
NB = 10240


def _hist_body(ei_hbm, iei_hbm, zeros_hbm, out_hbm,
               idx_v, ones_v, bins_sh, sem):
    c = lax.axis_index("c")
    s = lax.axis_index("s")

    def stage(src):
        @pl.when(s < 15)
        def _():
            pltpu.sync_copy(src.at[0, pl.ds(s * RPT, RPT)], idx_v)

        @pl.when(s == 15)
        def _():
            pltpu.sync_copy(src.at[0, pl.ds(15 * RPT, RLAST)],
                            idx_v.at[pl.ds(0, RLAST)])

    @pl.when(c == 0)
    def _():
        stage(ei_hbm)

    @pl.when(c == 1)
    def _():
        stage(iei_hbm)

    for i in range(CH // 16):
        ones_v[pl.ds(i * 16, 16)] = jnp.ones((16,), jnp.float32)

    @pl.when(s == 0)
    def _():
        pltpu.sync_copy(zeros_hbm, bins_sh)
    plsc.subcore_barrier()

    n_rows = jnp.where(s == 15, RLAST, RPT)

    def body(j, carry):
        pltpu.async_copy(ones_v, bins_sh.at[idx_v.at[j]], sem, add=True)
        return carry

    lax.fori_loop(0, n_rows, body, 0)

    def drain(j, carry):
        pltpu.make_async_copy(zeros_hbm.at[pl.ds(0, CH)], ones_v, sem).wait()
        return carry

    lax.fori_loop(0, n_rows, drain, 0)
    plsc.subcore_barrier()

    @pl.when(s == 0)
    def _():
        pltpu.sync_copy(bins_sh, out_hbm.at[c])


@functools.cache
def _hist_kernel():
    return pl.kernel(
        _hist_body,
        out_type=jax.ShapeDtypeStruct((2, NB), jnp.float32),
        mesh=plsc.VectorSubcoreMesh(core_axis_name="c", subcore_axis_name="s"),
        scratch_types=[
            pltpu.VMEM((RPT, CH), jnp.int32),
            pltpu.VMEM((CH,), jnp.float32),
            pltpu.VMEM_SHARED((NB,), jnp.float32),
            pltpu.SemaphoreType.DMA,
        ],
    )


_TDOT = (((1,), (1,)), ((), ()))


def _dense_body(x_ref, dege_ref, degi_ref, bat_ref,
                wu1_ref, bu1_ref, wu2_ref, bu2_ref,
                wu3_ref, bu3_ref, wu4_ref, bu4_ref,
                wf1_ref, bf1_ref, wf2_ref, bf2_ref, out_ref):
    x = x_ref[...]

    def conv(h, deg, wu_ref, bu_ref):
        wu = wu_ref[...]
        u = lax.dot_general(h, wu[:, :D], _TDOT,
                            preferred_element_type=jnp.float32)
        u = u + lax.dot_general(deg, wu[:, D:], _TDOT,
                                preferred_element_type=jnp.float32)
        u = u + bu_ref[...]
        u = jnp.where(u >= 0, u, 0.01 * u)
        u = u - jnp.max(u, axis=1, keepdims=True)
        e = jnp.exp(u)
        return e / jnp.sum(e, axis=1, keepdims=True)

    de = dege_ref[...]
    di = degi_ref[...]
    ext = conv(conv(x, de, wu1_ref, bu1_ref), de, wu2_ref, bu2_ref)
    itn = conv(conv(x, di, wu3_ref, bu3_ref), di, wu4_ref, bu4_ref)

    gids = lax.broadcasted_iota(jnp.int32, (G, 1), 0)
    pt = (bat_ref[...] == gids).astype(jnp.float32)
    cnt = jnp.dot(pt, jnp.ones((N, 1), jnp.float32),
                  preferred_element_type=jnp.float32)
    cnt = jnp.maximum(cnt, 1.0)
    ez = jnp.dot(pt, ext, preferred_element_type=jnp.float32) / cnt
    iz = jnp.dot(pt, itn, preferred_element_type=jnp.float32) / cnt

    wf1 = wf1_ref[...]
    z = (lax.dot_general(ez, wf1[:, :D], _TDOT,
                         preferred_element_type=jnp.float32)
         + lax.dot_general(iz, wf1[:, D:], _TDOT,
                           preferred_element_type=jnp.float32)
         + bf1_ref[...])
    z = jnp.maximum(z, 0.0)
    r = jnp.sum(z * wf2_ref[...], axis=1, keepdims=True)
    out_ref[...] = r + bf2_ref[0, 0]


def kernel(x, edge_index, edge_attr, internal_edge_index, internal_edge_attr,
           batch,
           wm_ext1, bm_ext1, wu_ext1, bu_ext1,
           wm_ext2, bm_ext2, wu_ext2, bu_ext2,
           wm_int1, bm_int1, wu_int1, bu_int1,
           wm_int2, bm_int2, wu_int2, bu_int2,
           w_fc1, b_fc1, w_fc2, b_fc2):
    ei3 = edge_index.reshape(2, ROWS, CH)
    iei3 = internal_edge_index.reshape(2, ROWS, CH)
    hists = _hist_kernel()(ei3, iei3, jnp.zeros((NB,), jnp.float32))
    dege = hists[0, :N]
    degi = hists[1, :N]

    return pl.pallas_call(
        _dense_body,
        out_shape=jax.ShapeDtypeStruct((G, 1), jnp.float32),
    )(x, dege.reshape(N, 1), degi.reshape(N, 1), batch.reshape(1, N),
      wu_ext1, bu_ext1, wu_ext2, bu_ext2,
      wu_int1, bu_int1, wu_int2, bu_int2,
      w_fc1, b_fc1, w_fc2, b_fc2.reshape(1, 1))
